# baseline (speedup 1.0000x reference)
import functools

import jax
import jax.numpy as jnp
from jax import lax
from jax.experimental import pallas as pl
from jax.experimental.pallas import tpu as pltpu
from jax.experimental.pallas import tpu_sc as plsc

B = 16384
T = 50
K = 6
NC = 2
NS = 16
L = 16
NW = NC * NS
RPW = B // NW
G = RPW // L
CHUNK = RPW * T
IDX_TOT = B * T
NUM_EMB = 300000
TBL_SLICE = 18752
NBLK = 4
BLK_G = G // NBLK
BLK_CH = CHUNK // (NBLK * 128)

_mesh = plsc.VectorSubcoreMesh(
    core_axis_name="c", subcore_axis_name="s", num_cores=NC, num_subcores=NS
)


@functools.partial(
    pl.kernel,
    out_type=jax.ShapeDtypeStruct((NW, L), jnp.float32),
    mesh=_mesh,
    scratch_types=[
        pltpu.VMEM((CHUNK,), jnp.int32),
        pltpu.VMEM((CHUNK,), jnp.float32),
        pltpu.VMEM((CHUNK,), jnp.int32),
        pltpu.VMEM((CHUNK,), jnp.float32),
        pltpu.VMEM((RPW,), jnp.float32),
        pltpu.VMEM((RPW,), jnp.float32),
        pltpu.VMEM((RPW,), jnp.float32),
        pltpu.VMEM((1, L), jnp.float32),
        pltpu.VMEM_SHARED((NUM_EMB,), jnp.float32),
    ] + [pltpu.SemaphoreType.DMA] * 4,
    compiler_params=pltpu.CompilerParams(needs_layout_passes=False),
)
def _sc_loss(team1_hbm, team2_hbm, res_hbm, emb_hbm, out_hbm,
             idx1_v, vals1_v, idx2_v, vals2_v, s1_v, s2_v, res_v,
             acc_v, tbl_s, *sems):
    cid = lax.axis_index("c")
    sid = lax.axis_index("s")
    wid = sid * NC + cid

    toff = sid * TBL_SLICE
    ic1, ic2 = [], []
    for t in range(T):
        ic1.append(pltpu.async_copy(
            team1_hbm.at[t, pl.ds(wid * RPW, RPW)],
            idx1_v.at[pl.ds(t * RPW, RPW)], sems[2]))
        ic2.append(pltpu.async_copy(
            team2_hbm.at[t, pl.ds(wid * RPW, RPW)],
            idx2_v.at[pl.ds(t * RPW, RPW)], sems[3]))
    pltpu.sync_copy(res_hbm.at[pl.ds(wid * RPW, RPW)], res_v)

    @pl.when(sid < NS - 1)
    def _():
        pltpu.sync_copy(emb_hbm.at[pl.ds(toff, TBL_SLICE)],
                        vals1_v.at[pl.ds(0, TBL_SLICE)])
        pltpu.sync_copy(vals1_v.at[pl.ds(0, TBL_SLICE)],
                        tbl_s.at[pl.ds(toff, TBL_SLICE)])

    @pl.when(sid == NS - 1)
    def _():
        last = NUM_EMB - (NS - 1) * TBL_SLICE
        pltpu.sync_copy(emb_hbm.at[pl.ds((NS - 1) * TBL_SLICE, last)],
                        vals1_v.at[pl.ds(0, last)])
        pltpu.sync_copy(vals1_v.at[pl.ds(0, last)],
                        tbl_s.at[pl.ds((NS - 1) * TBL_SLICE, last)])

    plsc.subcore_barrier()

    for d in ic1:
        d.wait()
    g1 = [pltpu.async_copy(tbl_s.at[idx1_v], vals1_v, sems[0])]
    for d in ic2:
        d.wait()
    g2 = [pltpu.async_copy(tbl_s.at[idx2_v], vals2_v, sems[1])]

    def score_all(vals_v, s_v):
        def g_body(g, carry):
            o = g * L
            m = [jnp.full((L,), -3.0e38, jnp.float32) for _ in range(K)]
            for t in range(T):
                v = vals_v[pl.ds(t * RPW + o, L)]
                for i in range(K):
                    hi = jnp.maximum(m[i], v)
                    v = jnp.minimum(m[i], v)
                    m[i] = hi
            s = m[0]
            for i in range(1, K):
                s = s + m[i]
            s_v[pl.ds(o, L)] = s * jnp.float32(1.0 / K)
            return carry
        lax.fori_loop(0, G, g_body, 0)

    for d in g1:
        d.wait()
    score_all(vals1_v, s1_v)
    for d in g2:
        d.wait()
    score_all(vals2_v, s2_v)

    def loss_body(g, acc):
        o = g * L
        d = s1_v[pl.ds(o, L)] - s2_v[pl.ds(o, L)]
        p = jnp.float32(1.0) / (jnp.float32(1.0) + jnp.exp(-d))
        return acc + jnp.abs(p * jnp.float32(2.0) - jnp.float32(1.0)
                             - res_v[pl.ds(o, L)])

    acc = lax.fori_loop(0, G, loss_body, jnp.zeros((L,), jnp.float32))
    acc_v[0, :] = acc
    pltpu.sync_copy(acc_v, out_hbm.at[pl.ds(wid, 1)])


def kernel(team_1, team_2, result, emb_weight):
    res = result.reshape(B)
    emb = emb_weight.reshape(-1)
    partials = _sc_loss(team_1.T, team_2.T, res, emb)
    return jnp.sum(partials) * jnp.float32(1.0 / B)


# speedup vs baseline: 1.0428x; 1.0428x over previous
import functools

import jax
import jax.numpy as jnp
from jax import lax
from jax.experimental import pallas as pl
from jax.experimental.pallas import tpu as pltpu
from jax.experimental.pallas import tpu_sc as plsc

B = 16384
T = 50
K = 6
NC = 2
NS = 16
L = 16
NW = NC * NS
RPW = B // NW
G = RPW // L
CHUNK = RPW * T
IDX_TOT = B * T
NUM_EMB = 300000
TBL_SLICE = 18752
NBLK = 4
BLK_G = G // NBLK
BLK_CH = CHUNK // (NBLK * 128)

_mesh = plsc.VectorSubcoreMesh(
    core_axis_name="c", subcore_axis_name="s", num_cores=NC, num_subcores=NS
)


@functools.partial(
    pl.kernel,
    out_type=jax.ShapeDtypeStruct((NW, L), jnp.float32),
    mesh=_mesh,
    scratch_types=[
        pltpu.VMEM((CHUNK,), jnp.int32),
        pltpu.VMEM((CHUNK,), jnp.float32),
        pltpu.VMEM((CHUNK,), jnp.int32),
        pltpu.VMEM((CHUNK,), jnp.float32),
        pltpu.VMEM((RPW,), jnp.float32),
        pltpu.VMEM((RPW,), jnp.float32),
        pltpu.VMEM((RPW,), jnp.float32),
        pltpu.VMEM((1, L), jnp.float32),
        pltpu.VMEM_SHARED((NUM_EMB,), jnp.float32),
    ] + [pltpu.SemaphoreType.DMA] * 4,
    compiler_params=pltpu.CompilerParams(needs_layout_passes=False),
)
def _sc_loss(team1_hbm, team2_hbm, res_hbm, emb_hbm, out_hbm,
             idx1_v, vals1_v, idx2_v, vals2_v, s1_v, s2_v, res_v,
             acc_v, tbl_s, *sems):
    cid = lax.axis_index("c")
    sid = lax.axis_index("s")
    wid = sid * NC + cid

    toff = sid * TBL_SLICE
    ic1, ic2 = [], []
    for t in range(T):
        ic1.append(pltpu.async_copy(
            team1_hbm.at[t, pl.ds(wid * RPW, RPW)],
            idx1_v.at[pl.ds(t * RPW, RPW)], sems[2]))
        ic2.append(pltpu.async_copy(
            team2_hbm.at[t, pl.ds(wid * RPW, RPW)],
            idx2_v.at[pl.ds(t * RPW, RPW)], sems[3]))
    pltpu.sync_copy(res_hbm.at[pl.ds(wid * RPW, RPW)], res_v)

    @pl.when(sid < NS - 1)
    def _():
        pltpu.sync_copy(emb_hbm.at[pl.ds(toff, TBL_SLICE)],
                        vals1_v.at[pl.ds(0, TBL_SLICE)])
        pltpu.sync_copy(vals1_v.at[pl.ds(0, TBL_SLICE)],
                        tbl_s.at[pl.ds(toff, TBL_SLICE)])

    @pl.when(sid == NS - 1)
    def _():
        last = NUM_EMB - (NS - 1) * TBL_SLICE
        pltpu.sync_copy(emb_hbm.at[pl.ds((NS - 1) * TBL_SLICE, last)],
                        vals1_v.at[pl.ds(0, last)])
        pltpu.sync_copy(vals1_v.at[pl.ds(0, last)],
                        tbl_s.at[pl.ds((NS - 1) * TBL_SLICE, last)])

    plsc.subcore_barrier()

    for d in ic1:
        d.wait()
    g1 = [pltpu.async_copy(tbl_s.at[idx1_v.at[pl.ds(t * RPW, RPW)]],
                           vals1_v.at[pl.ds(t * RPW, RPW)], sems[0])
          for t in range(T)]
    for d in ic2:
        d.wait()
    g2 = [pltpu.async_copy(tbl_s.at[idx2_v.at[pl.ds(t * RPW, RPW)]],
                           vals2_v.at[pl.ds(t * RPW, RPW)], sems[1])
          for t in range(T)]

    def score_all(vals_v, s_v):
        def g_body(g, carry):
            o = g * L
            m = [jnp.full((L,), -3.0e38, jnp.float32) for _ in range(K)]
            for t in range(T):
                v = vals_v[pl.ds(t * RPW + o, L)]
                for i in range(K):
                    hi = jnp.maximum(m[i], v)
                    v = jnp.minimum(m[i], v)
                    m[i] = hi
            s = m[0]
            for i in range(1, K):
                s = s + m[i]
            s_v[pl.ds(o, L)] = s * jnp.float32(1.0 / K)
            return carry
        lax.fori_loop(0, G, g_body, 0)

    for d in g1:
        d.wait()
    score_all(vals1_v, s1_v)
    for d in g2:
        d.wait()
    score_all(vals2_v, s2_v)

    def loss_body(g, acc):
        o = g * L
        d = s1_v[pl.ds(o, L)] - s2_v[pl.ds(o, L)]
        p = jnp.float32(1.0) / (jnp.float32(1.0) + jnp.exp(-d))
        return acc + jnp.abs(p * jnp.float32(2.0) - jnp.float32(1.0)
                             - res_v[pl.ds(o, L)])

    acc = lax.fori_loop(0, G, loss_body, jnp.zeros((L,), jnp.float32))
    acc_v[0, :] = acc
    pltpu.sync_copy(acc_v, out_hbm.at[pl.ds(wid, 1)])


def kernel(team_1, team_2, result, emb_weight):
    res = result.reshape(B)
    emb = emb_weight.reshape(-1)
    partials = _sc_loss(team_1.T, team_2.T, res, emb)
    return jnp.sum(partials) * jnp.float32(1.0 / B)
